# baseline (speedup 1.0000x reference)
import functools

import jax
import jax.numpy as jnp
from jax import lax
from jax.experimental import pallas as pl
from jax.experimental.pallas import tpu as pltpu
from jax.experimental.pallas import tpu_sc as plsc

N = 10000
E = 320000
B = 10
P = 1000
D = 128
C = 128

NPAD = 10112
TILES = 32
CH = 128
NCH = 84
NPH = NCH // 2
E_PAD = TILES * NCH * CH
ROWS_PER_TILE = NPAD // 16

_mesh = plsc.VectorSubcoreMesh(core_axis_name="c", subcore_axis_name="s")


def _sc_deg_body(dst_hbm, ones_hbm, zeros_hbm, out_hbm,
                 dst_ids, ones_v, acc, sem_a, sem_b):
    c = lax.axis_index("c")
    s = lax.axis_index("s")
    w = c * 16 + s
    base = s * ROWS_PER_TILE
    cz = pltpu.async_copy(zeros_hbm, acc.at[pl.ds(base, ROWS_PER_TILE)], sem_a)
    cd = pltpu.async_copy(dst_hbm.at[2 * w], dst_ids, sem_b)
    pltpu.sync_copy(ones_hbm, ones_v)
    cz.wait()
    cd.wait()
    plsc.subcore_barrier()

    def body(j, carry):
        pltpu.sync_copy(ones_v, acc.at[dst_ids.at[j]], add=True)
        return carry

    for p in range(2):
        if p:
            pltpu.async_copy(dst_hbm.at[2 * w + 1], dst_ids, sem_b).wait()
        lax.fori_loop(0, NPH, body, 0)
    plsc.subcore_barrier()
    pltpu.sync_copy(acc.at[pl.ds(base, ROWS_PER_TILE)],
                    out_hbm.at[c].at[pl.ds(base, ROWS_PER_TILE)])


_sc_deg = pl.kernel(
    _sc_deg_body,
    out_type=jax.ShapeDtypeStruct((2, NPAD, C), jnp.float32),
    mesh=_mesh,
    scratch_types=[
        pltpu.VMEM((NPH, CH), jnp.int32),
        pltpu.VMEM((CH, C), jnp.float32),
        pltpu.VMEM_SHARED((NPAD, C), jnp.float32),
        pltpu.SemaphoreType.DMA,
        pltpu.SemaphoreType.DMA,
    ],
)


def _sc_spmm_body(y_hbm, src_hbm, dst_hbm, zeros_hbm, out_hbm,
                  src_ids, dst_ids, rows_a, rows_b,
                  acc, sem_a, sem_b, sem_i):
    c = lax.axis_index("c")
    s = lax.axis_index("s")
    w = c * 16 + s
    base = s * ROWS_PER_TILE
    cz = pltpu.async_copy(zeros_hbm, acc.at[pl.ds(base, ROWS_PER_TILE)], sem_a)
    cz.wait()
    plsc.subcore_barrier()

    def body(i, carry):
        j = 2 * i
        pltpu.async_copy(y_hbm.at[src_ids.at[j + 1]], rows_b, sem_b)
        pltpu.make_async_copy(y_hbm.at[src_ids.at[j]], rows_a, sem_a).wait()
        pltpu.sync_copy(rows_a, acc.at[dst_ids.at[j]], add=True)

        @pl.when(j + 2 < NPH)
        def _():
            pltpu.async_copy(y_hbm.at[src_ids.at[j + 2]], rows_a, sem_a)

        pltpu.make_async_copy(y_hbm.at[src_ids.at[j + 1]], rows_b, sem_b).wait()
        pltpu.sync_copy(rows_b, acc.at[dst_ids.at[j + 1]], add=True)
        return carry

    for p in range(2):
        blk = 2 * w + p
        ci = pltpu.async_copy(src_hbm.at[blk], src_ids, sem_i)
        cd = pltpu.async_copy(dst_hbm.at[blk], dst_ids, sem_i)
        ci.wait()
        cd.wait()
        pltpu.async_copy(y_hbm.at[src_ids.at[0]], rows_a, sem_a)
        lax.fori_loop(0, NPH // 2, body, 0)

    plsc.subcore_barrier()
    pltpu.sync_copy(acc.at[pl.ds(base, ROWS_PER_TILE)],
                    out_hbm.at[c].at[pl.ds(base, ROWS_PER_TILE)])


_sc_spmm = pl.kernel(
    _sc_spmm_body,
    out_type=jax.ShapeDtypeStruct((2, NPAD, C), jnp.float32),
    mesh=_mesh,
    scratch_types=[
        pltpu.VMEM((NPH, CH), jnp.int32),
        pltpu.VMEM((NPH, CH), jnp.int32),
        pltpu.VMEM((CH, C), jnp.float32),
        pltpu.VMEM((CH, C), jnp.float32),
        pltpu.VMEM_SHARED((NPAD, C), jnp.float32),
        pltpu.SemaphoreType.DMA,
        pltpu.SemaphoreType.DMA,
        pltpu.SemaphoreType.DMA,
    ],
)


def _tc_head_body(deg_ref, x_ref, w_ref, dinv_ref, y_ref):
    deg = deg_ref[0, :, 0:1] + deg_ref[1, :, 0:1]
    rowid = lax.broadcasted_iota(jnp.int32, (NPAD, 1), 0)
    dinv = jnp.where(rowid < N, lax.rsqrt(jnp.maximum(deg, 1.0)), 0.0)
    dinv_ref[...] = dinv
    y_ref[...] = jnp.dot(x_ref[...], w_ref[...],
                         preferred_element_type=jnp.float32) * dinv


_tc_head = pl.pallas_call(
    _tc_head_body,
    out_shape=(
        jax.ShapeDtypeStruct((NPAD, 1), jnp.float32),
        jax.ShapeDtypeStruct((NPAD, C), jnp.float32),
    ),
)


def _tc_mid_body(acc_ref, dinv_ref, b_ref, w_ref, h_ref, y_ref):
    dinv = dinv_ref[...]
    a = acc_ref[0] + acc_ref[1]
    h = jnp.maximum(a * dinv + b_ref[...], 0.0)
    h_ref[...] = h
    y_ref[...] = jnp.dot(h, w_ref[...],
                         preferred_element_type=jnp.float32) * dinv


_tc_mid = pl.pallas_call(
    _tc_mid_body,
    out_shape=(
        jax.ShapeDtypeStruct((NPAD, C), jnp.float32),
        jax.ShapeDtypeStruct((NPAD, C), jnp.float32),
    ),
)


def _tc_tail_body(acc_ref, dinv_ref, b3_ref, h1_ref, h2_ref, pm_ref,
                  lw1_ref, lb1_ref, lw2_ref, lb2_ref,
                  lw3_ref, lb3_ref, lw4_ref, lb4_ref, out_ref):
    h3 = jnp.maximum((acc_ref[0] + acc_ref[1]) * dinv_ref[...] + b3_ref[...],
                     0.0)
    h = h1_ref[...] + h2_ref[...] + h3
    cols = lax.broadcasted_iota(jnp.int32, (B, NPAD), 1)
    rows = lax.broadcasted_iota(jnp.int32, (B, NPAD), 0)
    mask = jnp.where((cols // P) == rows,
                     jnp.broadcast_to(pm_ref[...], (B, NPAD)), 0.0)
    pooled = jnp.dot(mask, h, preferred_element_type=jnp.float32)
    z = jnp.maximum(jnp.dot(pooled, lw1_ref[...],
                            preferred_element_type=jnp.float32)
                    + lb1_ref[...], 0.0)
    z = jnp.maximum(jnp.dot(z, lw2_ref[...],
                            preferred_element_type=jnp.float32)
                    + lb2_ref[...], 0.0)
    z = jnp.maximum(jnp.dot(z, lw3_ref[...],
                            preferred_element_type=jnp.float32)
                    + lb3_ref[...], 0.0)
    out_ref[...] = (jnp.dot(z, lw4_ref[...],
                            preferred_element_type=jnp.float32)
                    + lb4_ref[...])


_tc_tail = pl.pallas_call(
    _tc_tail_body,
    out_shape=jax.ShapeDtypeStruct((B, 1), jnp.float32),
)


def kernel(x, edge_index, protein_mask, batch,
           W1, b1, W2, b2, W3, b3,
           lw1, lb1, lw2, lb2, lw3, lb3, lw4, lb4):
    del batch
    loops = jnp.arange(N, dtype=jnp.int32)
    n_pad_edges = E_PAD - E - N
    pad_ids = jnp.full((n_pad_edges,), N, jnp.int32)
    src = jnp.concatenate([edge_index[0], loops, pad_ids])
    dst = jnp.concatenate([edge_index[1], loops, pad_ids])
    src3 = src.reshape(2 * TILES, NPH, CH)
    dst3 = dst.reshape(2 * TILES, NPH, CH)

    x_pad = jnp.pad(x, ((0, NPAD - N), (0, 0)))
    zeros128 = jnp.zeros((ROWS_PER_TILE, C), jnp.float32)
    ones128 = jnp.ones((CH, C), jnp.float32)
    pm_flat = jnp.pad(protein_mask.reshape(1, N), ((0, 0), (0, NPAD - N)))

    deg2 = _sc_deg(dst3, ones128, zeros128)
    dinv, y1 = _tc_head(deg2, x_pad, W1)
    acc1 = _sc_spmm(y1, src3, dst3, zeros128)
    h1, y2 = _tc_mid(acc1, dinv, b1.reshape(1, C), W2)
    acc2 = _sc_spmm(y2, src3, dst3, zeros128)
    h2, y3 = _tc_mid(acc2, dinv, b2.reshape(1, C), W3)
    acc3 = _sc_spmm(y3, src3, dst3, zeros128)
    z = _tc_tail(acc3, dinv, b3.reshape(1, C), h1, h2, pm_flat,
                 lw1, lb1.reshape(1, -1), lw2, lb2.reshape(1, -1),
                 lw3, lb3.reshape(1, -1), lw4, lb4.reshape(1, -1))
    return z


# speedup vs baseline: 2.2357x; 2.2357x over previous
import functools

import jax
import jax.numpy as jnp
from jax import lax
from jax.experimental import pallas as pl
from jax.experimental.pallas import tpu as pltpu
from jax.experimental.pallas import tpu_sc as plsc

N = 10000
E = 320000
B = 10
P = 1000
D = 128
C = 128

NPAD = 10112
TILES = 32
CH = 128
NCH = 81
E_PAD = TILES * NCH * CH
ROWS_PER_TILE = NPAD // 16

_mesh = plsc.VectorSubcoreMesh(core_axis_name="c", subcore_axis_name="s")


def _sc_deg_body(dst_hbm, ones_hbm, zeros_hbm, out_hbm,
                 dst_ids, ones_v, acc, sem):
    c = lax.axis_index("c")
    s = lax.axis_index("s")
    w = c * 16 + s
    base = s * ROWS_PER_TILE
    pltpu.sync_copy(zeros_hbm, acc.at[pl.ds(base, ROWS_PER_TILE)])
    pltpu.sync_copy(ones_hbm, ones_v)
    pltpu.async_copy(dst_hbm.at[w], dst_ids, sem).wait()
    plsc.subcore_barrier()

    def body(j, carry):
        pltpu.sync_copy(ones_v, acc.at[dst_ids.at[j]], add=True)
        return carry

    lax.fori_loop(0, NCH, body, 0)
    plsc.subcore_barrier()
    pltpu.sync_copy(acc.at[pl.ds(base, ROWS_PER_TILE)],
                    out_hbm.at[c].at[pl.ds(base, ROWS_PER_TILE)])


_sc_deg = pl.kernel(
    _sc_deg_body,
    out_type=jax.ShapeDtypeStruct((2, NPAD, C), jnp.float32),
    mesh=_mesh,
    scratch_types=[
        pltpu.VMEM((NCH, CH), jnp.int32),
        pltpu.VMEM((CH, C), jnp.float32),
        pltpu.VMEM_SHARED((NPAD, C), jnp.float32),
        pltpu.SemaphoreType.DMA,
    ],
)


def _sc_spmm_body(y_hbm, src_hbm, dst_hbm, zeros_hbm, out_hbm,
                  src_ids, dst_ids, rows, acc, sem):
    c = lax.axis_index("c")
    s = lax.axis_index("s")
    w = c * 16 + s
    base = s * ROWS_PER_TILE
    pltpu.sync_copy(zeros_hbm, acc.at[pl.ds(base, ROWS_PER_TILE)])
    pltpu.async_copy(src_hbm.at[w], src_ids, sem).wait()
    pltpu.async_copy(dst_hbm.at[w], dst_ids, sem).wait()
    plsc.subcore_barrier()

    def body(j, carry):
        pltpu.async_copy(y_hbm.at[src_ids.at[j]], rows, sem).wait()
        pltpu.sync_copy(rows, acc.at[dst_ids.at[j]], add=True)
        return carry

    lax.fori_loop(0, NCH, body, 0)
    plsc.subcore_barrier()
    pltpu.sync_copy(acc.at[pl.ds(base, ROWS_PER_TILE)],
                    out_hbm.at[c].at[pl.ds(base, ROWS_PER_TILE)])


_sc_spmm = pl.kernel(
    _sc_spmm_body,
    out_type=jax.ShapeDtypeStruct((2, NPAD, C), jnp.float32),
    mesh=_mesh,
    scratch_types=[
        pltpu.VMEM((NCH, CH), jnp.int32),
        pltpu.VMEM((NCH, CH), jnp.int32),
        pltpu.VMEM((CH, C), jnp.float32),
        pltpu.VMEM_SHARED((NPAD, C), jnp.float32),
        pltpu.SemaphoreType.DMA,
    ],
)


def _tc_head_body(deg_ref, x_ref, w_ref, dinv_ref, y_ref):
    deg = deg_ref[0, :, 0:1] + deg_ref[1, :, 0:1]
    rowid = lax.broadcasted_iota(jnp.int32, (NPAD, 1), 0)
    dinv = jnp.where(rowid < N, lax.rsqrt(jnp.maximum(deg, 1.0)), 0.0)
    dinv_ref[...] = dinv
    y_ref[...] = jnp.dot(x_ref[...], w_ref[...],
                         preferred_element_type=jnp.float32) * dinv


_tc_head = pl.pallas_call(
    _tc_head_body,
    out_shape=(
        jax.ShapeDtypeStruct((NPAD, 1), jnp.float32),
        jax.ShapeDtypeStruct((NPAD, C), jnp.float32),
    ),
)


def _tc_mid_body(acc_ref, dinv_ref, b_ref, w_ref, h_ref, y_ref):
    dinv = dinv_ref[...]
    a = acc_ref[0] + acc_ref[1]
    h = jnp.maximum(a * dinv + b_ref[...], 0.0)
    h_ref[...] = h
    y_ref[...] = jnp.dot(h, w_ref[...],
                         preferred_element_type=jnp.float32) * dinv


_tc_mid = pl.pallas_call(
    _tc_mid_body,
    out_shape=(
        jax.ShapeDtypeStruct((NPAD, C), jnp.float32),
        jax.ShapeDtypeStruct((NPAD, C), jnp.float32),
    ),
)


def _tc_tail_body(acc_ref, dinv_ref, b3_ref, h1_ref, h2_ref, pm_ref,
                  lw1_ref, lb1_ref, lw2_ref, lb2_ref,
                  lw3_ref, lb3_ref, lw4_ref, lb4_ref, out_ref):
    h3 = jnp.maximum((acc_ref[0] + acc_ref[1]) * dinv_ref[...] + b3_ref[...],
                     0.0)
    h = h1_ref[...] + h2_ref[...] + h3
    cols = lax.broadcasted_iota(jnp.int32, (B, NPAD), 1)
    rows = lax.broadcasted_iota(jnp.int32, (B, NPAD), 0)
    mask = jnp.where((cols // P) == rows,
                     jnp.broadcast_to(pm_ref[...], (B, NPAD)), 0.0)
    pooled = jnp.dot(mask, h, preferred_element_type=jnp.float32)
    z = jnp.maximum(jnp.dot(pooled, lw1_ref[...],
                            preferred_element_type=jnp.float32)
                    + lb1_ref[...], 0.0)
    z = jnp.maximum(jnp.dot(z, lw2_ref[...],
                            preferred_element_type=jnp.float32)
                    + lb2_ref[...], 0.0)
    z = jnp.maximum(jnp.dot(z, lw3_ref[...],
                            preferred_element_type=jnp.float32)
                    + lb3_ref[...], 0.0)
    out_ref[...] = (jnp.dot(z, lw4_ref[...],
                            preferred_element_type=jnp.float32)
                    + lb4_ref[...])


_tc_tail = pl.pallas_call(
    _tc_tail_body,
    out_shape=jax.ShapeDtypeStruct((B, 1), jnp.float32),
)


def kernel(x, edge_index, protein_mask, batch,
           W1, b1, W2, b2, W3, b3,
           lw1, lb1, lw2, lb2, lw3, lb3, lw4, lb4):
    del batch
    loops = jnp.arange(N, dtype=jnp.int32)
    n_pad_edges = E_PAD - E - N
    pad_ids = jnp.full((n_pad_edges,), N, jnp.int32)
    src = jnp.concatenate([edge_index[0], loops, pad_ids])
    dst = jnp.concatenate([edge_index[1], loops, pad_ids])
    src3 = src.reshape(TILES, NCH, CH)
    dst3 = dst.reshape(TILES, NCH, CH)

    x_pad = jnp.pad(x, ((0, NPAD - N), (0, 0)))
    zeros128 = jnp.zeros((ROWS_PER_TILE, C), jnp.float32)
    ones128 = jnp.ones((CH, C), jnp.float32)
    pm_flat = jnp.pad(protein_mask.reshape(1, N), ((0, 0), (0, NPAD - N)))

    deg2 = _sc_deg(dst3, ones128, zeros128)
    dinv, y1 = _tc_head(deg2, x_pad, W1)
    acc1 = _sc_spmm(y1, src3, dst3, zeros128)
    h1, y2 = _tc_mid(acc1, dinv, b1.reshape(1, C), W2)
    acc2 = _sc_spmm(y2, src3, dst3, zeros128)
    h2, y3 = _tc_mid(acc2, dinv, b2.reshape(1, C), W3)
    acc3 = _sc_spmm(y3, src3, dst3, zeros128)
    z = _tc_tail(acc3, dinv, b3.reshape(1, C), h1, h2, pm_flat,
                 lw1, lb1.reshape(1, -1), lw2, lb2.reshape(1, -1),
                 lw3, lb3.reshape(1, -1), lw4, lb4.reshape(1, -1))
    return z
